# Initial kernel scaffold; baseline (speedup 1.0000x reference)
import jax
import jax.numpy as jnp
from jax.experimental import pallas as pl


def kernel(feats, src_nodes0, dstsrc2src0_1, dstsrc2src0_2, dstsrc2dst0_1, dstsrc2dst0_2, dif_mat0_1, dif_mat0_2, src_nodes1, dstsrc2src1_1, dstsrc2src1_2, dstsrc2dst1_1, dstsrc2dst1_2, dif_mat1_1, dif_mat1_2, w_agg1, w_agg2, W1, b1, W2, b2, W3, b3, W4, b4, W5, b5):
    raise NotImplementedError("write your pallas kernel here")


# speedup vs baseline: 1.4052x; 1.4052x over previous
import functools

import jax
import jax.numpy as jnp
from jax import lax
from jax.experimental import pallas as pl
from jax.experimental.pallas import tpu as pltpu
from jax.experimental.pallas import tpu_sc as plsc

F32 = jnp.float32

_NC, _NS = 2, 16
_NW = _NC * _NS


def _sc_gather(table, idx, chunk):
    b = idx.shape[0]
    d = table.shape[1]
    assert b % (8 * _NW) == 0
    b_per_w = b // _NW
    assert b_per_w % chunk == 0 and chunk % 8 == 0
    n_chunks = b_per_w // chunk
    mesh = plsc.VectorSubcoreMesh(core_axis_name="c", subcore_axis_name="s")

    @functools.partial(
        pl.kernel,
        out_type=jax.ShapeDtypeStruct((b, d), table.dtype),
        mesh=mesh,
        scratch_types=[
            pltpu.VMEM((chunk,), jnp.int32),
            pltpu.VMEM((chunk, d), table.dtype),
            pltpu.SemaphoreType.DMA,
        ],
    )
    def gather_kernel(table_hbm, idx_hbm, out_hbm, idx_v, rows_v, sem):
        wid = lax.axis_index("s") * _NC + lax.axis_index("c")
        base = wid * b_per_w

        @pl.loop(0, n_chunks)
        def _(c):
            off = base + c * chunk
            pltpu.sync_copy(idx_hbm.at[pl.ds(off, chunk)], idx_v)
            pltpu.async_copy(table_hbm.at[idx_v], rows_v, sem).wait()
            pltpu.sync_copy(rows_v, out_hbm.at[pl.ds(off, chunk)])

    return gather_kernel(table, idx)


def _agg_layer(dif, gathered, w, relu, bm, bk):
    m_total, s_total = dif.shape
    d = gathered.shape[1]
    grid = (m_total // bm, s_total // bk)
    n_k = grid[1]
    dst_block0 = s_total // bm

    def body(dif_ref, src_ref, dst_ref, wt_ref, wb_ref, o_ref, acc_ref):
        k = pl.program_id(1)

        @pl.when(k == 0)
        def _():
            acc_ref[...] = jnp.zeros_like(acc_ref)

        acc_ref[...] += jnp.dot(dif_ref[...], src_ref[...],
                                preferred_element_type=F32)

        @pl.when(k == n_k - 1)
        def _():
            o = (jnp.dot(acc_ref[...], wt_ref[...], preferred_element_type=F32)
                 + jnp.dot(dst_ref[...], wb_ref[...], preferred_element_type=F32))
            o_ref[...] = jnp.maximum(o, 0.0) if relu else o

    return pl.pallas_call(
        body,
        grid=grid,
        in_specs=[
            pl.BlockSpec((bm, bk), lambda m, k: (m, k)),
            pl.BlockSpec((bk, d), lambda m, k: (k, 0)),
            pl.BlockSpec((bm, d), lambda m, k: (dst_block0 + m, 0)),
            pl.BlockSpec((d, d), lambda m, k: (0, 0)),
            pl.BlockSpec((d, d), lambda m, k: (1, 0)),
        ],
        out_specs=pl.BlockSpec((bm, d), lambda m, k: (m, 0)),
        out_shape=jax.ShapeDtypeStruct((m_total, d), F32),
        scratch_shapes=[pltpu.VMEM((bm, d), F32)],
        compiler_params=pltpu.CompilerParams(
            dimension_semantics=("parallel", "arbitrary")),
    )(dif, gathered, gathered, w, w)


def _mlp_head(x0, x1, W1, b1, W2, b2, W3, b3, W4, b4, W5, b5):
    n = x0.shape[0]
    d = x0.shape[1]

    def body(x0_ref, x1_ref, w1t_ref, w1b_ref, b1_ref, w2_ref, b2_ref,
             w3_ref, b3_ref, w4_ref, b4_ref, w5_ref, b5_ref, o_ref):
        z = (jnp.dot(x0_ref[...], w1t_ref[...], preferred_element_type=F32)
             + jnp.dot(x1_ref[...], w1b_ref[...], preferred_element_type=F32)
             + b1_ref[...])
        z = jnp.maximum(z, 0.0)
        z = jnp.maximum(jnp.dot(z, w2_ref[...], preferred_element_type=F32)
                        + b2_ref[...], 0.0)
        z = jnp.maximum(jnp.dot(z, w3_ref[...], preferred_element_type=F32)
                        + b3_ref[...], 0.0)
        z = jnp.maximum(jnp.dot(z, w4_ref[...], preferred_element_type=F32)
                        + b4_ref[...], 0.0)
        o_ref[...] = (jnp.dot(z, w5_ref[...], preferred_element_type=F32)
                      + b5_ref[...])

    specs = [
        pl.BlockSpec((n, d), lambda i: (0, 0)),
        pl.BlockSpec((n, d), lambda i: (0, 0)),
        pl.BlockSpec((d, 128), lambda i: (0, 0)),
        pl.BlockSpec((d, 128), lambda i: (1, 0)),
        pl.BlockSpec((1, 128), lambda i: (0, 0)),
        pl.BlockSpec((128, 64), lambda i: (0, 0)),
        pl.BlockSpec((1, 64), lambda i: (0, 0)),
        pl.BlockSpec((64, 32), lambda i: (0, 0)),
        pl.BlockSpec((1, 32), lambda i: (0, 0)),
        pl.BlockSpec((32, 8), lambda i: (0, 0)),
        pl.BlockSpec((1, 8), lambda i: (0, 0)),
        pl.BlockSpec((8, 1), lambda i: (0, 0)),
        pl.BlockSpec((1, 1), lambda i: (0, 0)),
    ]
    return pl.pallas_call(
        body,
        grid=(1,),
        in_specs=specs,
        out_specs=pl.BlockSpec((n, 1), lambda i: (0, 0)),
        out_shape=jax.ShapeDtypeStruct((n, 1), F32),
    )(x0, x1, W1, W1, b1.reshape(1, -1), W2, b2.reshape(1, -1),
      W3, b3.reshape(1, -1), W4, b4.reshape(1, -1), W5, b5.reshape(1, -1))


def _branch_layer1(feats, src_nodes, s2s, s2d, dif, w_agg1):
    idx = jnp.concatenate([jnp.take(src_nodes, s2s, axis=0),
                           jnp.take(src_nodes, s2d, axis=0)])
    g = _sc_gather(feats, idx, chunk=192)
    return _agg_layer(dif, g, w_agg1, relu=True, bm=512, bk=2048)


def _branch_layer2(h1, s2s, s2d, dif, w_agg2):
    idx = jnp.concatenate([s2s, s2d])
    g = _sc_gather(h1, idx, chunk=160)
    return _agg_layer(dif, g, w_agg2, relu=False, bm=1024, bk=2048)


def kernel(feats, src_nodes0, dstsrc2src0_1, dstsrc2src0_2, dstsrc2dst0_1,
           dstsrc2dst0_2, dif_mat0_1, dif_mat0_2, src_nodes1, dstsrc2src1_1,
           dstsrc2src1_2, dstsrc2dst1_1, dstsrc2dst1_2, dif_mat1_1,
           dif_mat1_2, w_agg1, w_agg2, W1, b1, W2, b2, W3, b3, W4, b4, W5,
           b5):
    h1_0 = _branch_layer1(feats, src_nodes0, dstsrc2src0_2, dstsrc2dst0_2,
                          dif_mat0_2, w_agg1)
    h1_1 = _branch_layer1(feats, src_nodes1, dstsrc2src1_2, dstsrc2dst1_2,
                          dif_mat1_2, w_agg1)
    x = _branch_layer2(h1_0, dstsrc2src0_1, dstsrc2dst0_1, dif_mat0_1, w_agg2)
    y = _branch_layer2(h1_1, dstsrc2src1_1, dstsrc2dst1_1, dif_mat1_1, w_agg2)
    return _mlp_head(x, y, W1, b1, W2, b2, W3, b3, W4, b4, W5, b5)


# speedup vs baseline: 1.4054x; 1.0001x over previous
import functools

import jax
import jax.numpy as jnp
from jax import lax
from jax.experimental import pallas as pl
from jax.experimental.pallas import tpu as pltpu
from jax.experimental.pallas import tpu_sc as plsc

F32 = jnp.float32

_NC, _NS = 2, 16
_NW = _NC * _NS


def _sc_gather(table, idx, chunk):
    b = idx.shape[0]
    d = table.shape[1]
    assert b % (8 * _NW) == 0
    b_per_w = b // _NW
    assert b_per_w % chunk == 0 and chunk % 8 == 0
    n_chunks = b_per_w // chunk
    mesh = plsc.VectorSubcoreMesh(core_axis_name="c", subcore_axis_name="s")

    @functools.partial(
        pl.kernel,
        out_type=jax.ShapeDtypeStruct((b, d), table.dtype),
        mesh=mesh,
        scratch_types=[
            pltpu.VMEM((chunk,), jnp.int32),
            pltpu.VMEM((chunk, d), table.dtype),
            pltpu.SemaphoreType.DMA,
        ],
    )
    def gather_kernel(table_hbm, idx_hbm, out_hbm, idx_v, rows_v, sem):
        wid = lax.axis_index("s") * _NC + lax.axis_index("c")
        base = wid * b_per_w

        @pl.loop(0, n_chunks)
        def _(c):
            off = base + c * chunk
            pltpu.sync_copy(idx_hbm.at[pl.ds(off, chunk)], idx_v)
            pltpu.async_copy(table_hbm.at[idx_v], rows_v, sem).wait()
            pltpu.sync_copy(rows_v, out_hbm.at[pl.ds(off, chunk)])

    return gather_kernel(table, idx)


def _agg_layer(dif, gathered, w, relu, bm, bk):
    m_total, s_total = dif.shape
    d = gathered.shape[1]
    grid = (m_total // bm, s_total // bk)
    n_k = grid[1]
    dst_block0 = s_total // bm

    def body(dif_ref, src_ref, dst_ref, wt_ref, wb_ref, o_ref, acc_ref):
        k = pl.program_id(1)

        @pl.when(k == 0)
        def _():
            acc_ref[...] = jnp.zeros_like(acc_ref)

        acc_ref[...] += jnp.dot(dif_ref[...].astype(jnp.bfloat16),
                                src_ref[...].astype(jnp.bfloat16),
                                preferred_element_type=F32)

        @pl.when(k == n_k - 1)
        def _():
            o = (jnp.dot(acc_ref[...], wt_ref[...], preferred_element_type=F32)
                 + jnp.dot(dst_ref[...], wb_ref[...], preferred_element_type=F32))
            o_ref[...] = jnp.maximum(o, 0.0) if relu else o

    return pl.pallas_call(
        body,
        grid=grid,
        in_specs=[
            pl.BlockSpec((bm, bk), lambda m, k: (m, k)),
            pl.BlockSpec((bk, d), lambda m, k: (k, 0)),
            pl.BlockSpec((bm, d), lambda m, k: (dst_block0 + m, 0)),
            pl.BlockSpec((d, d), lambda m, k: (0, 0)),
            pl.BlockSpec((d, d), lambda m, k: (1, 0)),
        ],
        out_specs=pl.BlockSpec((bm, d), lambda m, k: (m, 0)),
        out_shape=jax.ShapeDtypeStruct((m_total, d), F32),
        scratch_shapes=[pltpu.VMEM((bm, d), F32)],
        compiler_params=pltpu.CompilerParams(
            dimension_semantics=("parallel", "arbitrary")),
    )(dif, gathered, gathered, w, w)


def _mlp_head(x0, x1, W1, b1, W2, b2, W3, b3, W4, b4, W5, b5):
    n = x0.shape[0]
    d = x0.shape[1]

    def body(x0_ref, x1_ref, w1t_ref, w1b_ref, b1_ref, w2_ref, b2_ref,
             w3_ref, b3_ref, w4_ref, b4_ref, w5_ref, b5_ref, o_ref):
        z = (jnp.dot(x0_ref[...], w1t_ref[...], preferred_element_type=F32)
             + jnp.dot(x1_ref[...], w1b_ref[...], preferred_element_type=F32)
             + b1_ref[...])
        z = jnp.maximum(z, 0.0)
        z = jnp.maximum(jnp.dot(z, w2_ref[...], preferred_element_type=F32)
                        + b2_ref[...], 0.0)
        z = jnp.maximum(jnp.dot(z, w3_ref[...], preferred_element_type=F32)
                        + b3_ref[...], 0.0)
        z = jnp.maximum(jnp.dot(z, w4_ref[...], preferred_element_type=F32)
                        + b4_ref[...], 0.0)
        o_ref[...] = (jnp.dot(z, w5_ref[...], preferred_element_type=F32)
                      + b5_ref[...])

    specs = [
        pl.BlockSpec((n, d), lambda i: (0, 0)),
        pl.BlockSpec((n, d), lambda i: (0, 0)),
        pl.BlockSpec((d, 128), lambda i: (0, 0)),
        pl.BlockSpec((d, 128), lambda i: (1, 0)),
        pl.BlockSpec((1, 128), lambda i: (0, 0)),
        pl.BlockSpec((128, 64), lambda i: (0, 0)),
        pl.BlockSpec((1, 64), lambda i: (0, 0)),
        pl.BlockSpec((64, 32), lambda i: (0, 0)),
        pl.BlockSpec((1, 32), lambda i: (0, 0)),
        pl.BlockSpec((32, 8), lambda i: (0, 0)),
        pl.BlockSpec((1, 8), lambda i: (0, 0)),
        pl.BlockSpec((8, 1), lambda i: (0, 0)),
        pl.BlockSpec((1, 1), lambda i: (0, 0)),
    ]
    return pl.pallas_call(
        body,
        grid=(1,),
        in_specs=specs,
        out_specs=pl.BlockSpec((n, 1), lambda i: (0, 0)),
        out_shape=jax.ShapeDtypeStruct((n, 1), F32),
    )(x0, x1, W1, W1, b1.reshape(1, -1), W2, b2.reshape(1, -1),
      W3, b3.reshape(1, -1), W4, b4.reshape(1, -1), W5, b5.reshape(1, -1))


def _branch_layer1(feats, src_nodes, s2s, s2d, dif, w_agg1):
    idx = jnp.concatenate([jnp.take(src_nodes, s2s, axis=0),
                           jnp.take(src_nodes, s2d, axis=0)])
    g = _sc_gather(feats, idx, chunk=192)
    return _agg_layer(dif, g, w_agg1, relu=True, bm=512, bk=2048)


def _branch_layer2(h1, s2s, s2d, dif, w_agg2):
    idx = jnp.concatenate([s2s, s2d])
    g = _sc_gather(h1, idx, chunk=160)
    return _agg_layer(dif, g, w_agg2, relu=False, bm=1024, bk=2048)


def kernel(feats, src_nodes0, dstsrc2src0_1, dstsrc2src0_2, dstsrc2dst0_1,
           dstsrc2dst0_2, dif_mat0_1, dif_mat0_2, src_nodes1, dstsrc2src1_1,
           dstsrc2src1_2, dstsrc2dst1_1, dstsrc2dst1_2, dif_mat1_1,
           dif_mat1_2, w_agg1, w_agg2, W1, b1, W2, b2, W3, b3, W4, b4, W5,
           b5):
    h1_0 = _branch_layer1(feats, src_nodes0, dstsrc2src0_2, dstsrc2dst0_2,
                          dif_mat0_2, w_agg1)
    h1_1 = _branch_layer1(feats, src_nodes1, dstsrc2src1_2, dstsrc2dst1_2,
                          dif_mat1_2, w_agg1)
    x = _branch_layer2(h1_0, dstsrc2src0_1, dstsrc2dst0_1, dif_mat0_1, w_agg2)
    y = _branch_layer2(h1_1, dstsrc2src1_1, dstsrc2dst1_1, dif_mat1_1, w_agg2)
    return _mlp_head(x, y, W1, b1, W2, b2, W3, b3, W4, b4, W5, b5)
